# Initial kernel scaffold; baseline (speedup 1.0000x reference)
import functools

import jax
import jax.numpy as jnp
import numpy as np
from jax import lax
from jax.experimental import pallas as pl
from jax.experimental.pallas import tpu as pltpu
from jax.experimental.pallas import tpu_sc as plsc

CUTOFF = 10.0
SHIFT = float(np.log(2.0))

NC = 2
NS = 16
NW = NC * NS

K = 80


def _ssp(x):
    return jax.nn.softplus(x) - SHIFT


def _filters_body(ea_ref, el_ref, w1_ref, b1_ref, w2_ref, b2_ref, out_ref):
    ea = ea_ref[...]
    t = jnp.dot(ea, w1_ref[0], preferred_element_type=jnp.float32) + b1_ref[...]
    t = _ssp(t)
    w = jnp.dot(t, w2_ref[0], preferred_element_type=jnp.float32) + b2_ref[...]
    el = el_ref[...]
    c = 0.5 * (jnp.cos(el * (np.pi / CUTOFF)) + 1.0)
    c = c * (el <= CUTOFF).astype(jnp.float32) * (el >= 0.0).astype(jnp.float32)
    out_ref[0] = w * c


def _precompute_filters(edge_attr, edge_length, mlp_w1, mlp_b1, mlp_w2, mlp_b2):
    L, D_EDGE, D = mlp_w1.shape
    E = edge_attr.shape[0]
    BE = 2000
    el2 = edge_length.reshape(E, 1)
    grid = (L, E // BE)
    return pl.pallas_call(
        _filters_body,
        grid=grid,
        in_specs=[
            pl.BlockSpec((BE, D_EDGE), lambda l, e: (e, 0)),
            pl.BlockSpec((BE, 1), lambda l, e: (e, 0)),
            pl.BlockSpec((1, D_EDGE, D), lambda l, e: (l, 0, 0)),
            pl.BlockSpec((1, D), lambda l, e: (l, 0)),
            pl.BlockSpec((1, D, D), lambda l, e: (l, 0, 0)),
            pl.BlockSpec((1, D), lambda l, e: (l, 0)),
        ],
        out_specs=pl.BlockSpec((1, BE, D), lambda l, e: (l, e, 0)),
        out_shape=jax.ShapeDtypeStruct((L, E, D), jnp.float32),
    )(edge_attr, el2, mlp_w1, mlp_b1, mlp_w2, mlp_b2)


def _matmul_body(x_ref, w_ref, o_ref):
    o_ref[...] = jnp.dot(x_ref[...], w_ref[...], preferred_element_type=jnp.float32)


def _tc_matmul(x, w):
    n, d = x.shape
    BN = 2000
    return pl.pallas_call(
        _matmul_body,
        grid=(n // BN,),
        in_specs=[
            pl.BlockSpec((BN, d), lambda i: (i, 0)),
            pl.BlockSpec((d, d), lambda i: (0, 0)),
        ],
        out_specs=pl.BlockSpec((BN, d), lambda i: (i, 0)),
        out_shape=jax.ShapeDtypeStruct((n, d), jnp.float32),
    )(x, w)


def _update_body(aggp_ref, h_ref, l2w_ref, l2b_ref, lw_ref, lb_ref, l1n_ref,
                 hn_ref, x1_ref):
    agg = aggp_ref[0] + aggp_ref[1]
    t = jnp.dot(agg, l2w_ref[...], preferred_element_type=jnp.float32) + l2b_ref[...]
    t = _ssp(t)
    out = jnp.dot(t, lw_ref[...], preferred_element_type=jnp.float32) + lb_ref[...]
    hn = h_ref[...] + out
    hn_ref[...] = hn
    x1_ref[...] = jnp.dot(hn, l1n_ref[...], preferred_element_type=jnp.float32)


def _tc_update(aggp, h, l2w, l2b, lw, lb, l1n):
    n, d = h.shape
    BN = 2000
    return pl.pallas_call(
        _update_body,
        grid=(n // BN,),
        in_specs=[
            pl.BlockSpec((2, BN, d), lambda i: (0, i, 0)),
            pl.BlockSpec((BN, d), lambda i: (i, 0)),
            pl.BlockSpec((d, d), lambda i: (0, 0)),
            pl.BlockSpec((1, d), lambda i: (0, 0)),
            pl.BlockSpec((d, d), lambda i: (0, 0)),
            pl.BlockSpec((1, d), lambda i: (0, 0)),
            pl.BlockSpec((d, d), lambda i: (0, 0)),
        ],
        out_specs=[
            pl.BlockSpec((BN, d), lambda i: (i, 0)),
            pl.BlockSpec((BN, d), lambda i: (i, 0)),
        ],
        out_shape=[
            jax.ShapeDtypeStruct((n, d), jnp.float32),
            jax.ShapeDtypeStruct((n, d), jnp.float32),
        ],
    )(aggp, h, l2w, l2b, lw, lb, l1n)


def _sc_message(x1, w, src, dst):
    n, d = x1.shape
    e = src.shape[0]
    epw = e // NW
    chunks = epw // K
    rpw = n // NS
    nvec = d // 16

    mesh = plsc.VectorSubcoreMesh(core_axis_name="c", subcore_axis_name="s")

    @functools.partial(
        pl.kernel,
        out_type=jax.ShapeDtypeStruct((NC, n, d), jnp.float32),
        mesh=mesh,
        scratch_types=[
            pltpu.VMEM((K,), jnp.int32),
            pltpu.VMEM((K,), jnp.int32),
            pltpu.VMEM((K, d), jnp.float32),
            pltpu.VMEM((K, d), jnp.float32),
            pltpu.VMEM((n // NS, d), jnp.float32),
            pltpu.VMEM_SHARED((n, d), jnp.float32),
            pltpu.SemaphoreType.DMA,
        ],
    )
    def launch(x1_hbm, w_hbm, src_hbm, dst_hbm, out_hbm,
               idxs_v, idxd_v, w_v, x_v, z_v, acc_sh, sem):
        c = lax.axis_index("c")
        s = lax.axis_index("s")
        wid = s * NC + c
        base0 = wid * epw
        row0 = s * rpw

        def _zero(i, _):
            for j in range(nvec):
                z_v[i, pl.ds(j * 16, 16)] = jnp.zeros((16,), jnp.float32)
            return None
        lax.fori_loop(0, rpw, _zero, None)

        pltpu.sync_copy(z_v, acc_sh.at[pl.ds(row0, rpw)])
        plsc.subcore_barrier()

        def _edge_chunk(g, _):
            base = base0 + g * K
            pltpu.sync_copy(src_hbm.at[pl.ds(base, K)], idxs_v)
            pltpu.sync_copy(dst_hbm.at[pl.ds(base, K)], idxd_v)
            pltpu.async_copy(x1_hbm.at[idxs_v], x_v, sem).wait()
            pltpu.sync_copy(w_hbm.at[pl.ds(base, K)], w_v)

            def _mul(i, _):
                for j in range(nvec):
                    sl = pl.ds(j * 16, 16)
                    x_v[i, sl] = x_v[i, sl] * w_v[i, sl]
                return None
            lax.fori_loop(0, K, _mul, None)

            pltpu.sync_copy(x_v, acc_sh.at[idxd_v], add=True)
            return None
        lax.fori_loop(0, chunks, _edge_chunk, None)

        plsc.subcore_barrier()
        pltpu.sync_copy(acc_sh.at[pl.ds(row0, rpw)],
                        out_hbm.at[c, pl.ds(row0, rpw)])

    return launch(x1, w, src, dst)


def kernel(z, edge_index, edge_length, edge_attr, mlp_w1, mlp_b1, mlp_w2,
           mlp_b2, lin1_w, lin2_w, lin2_b, lin_w, lin_b):
    L = mlp_w1.shape[0]
    src = edge_index[0]
    dst = edge_index[1]

    w_all = _precompute_filters(edge_attr, edge_length, mlp_w1, mlp_b1,
                                mlp_w2, mlp_b2)

    h = z
    x1 = _tc_matmul(z, lin1_w[0])
    for i in range(L):
        aggp = _sc_message(x1, w_all[i], src, dst)
        l1n = lin1_w[(i + 1) % L]
        h, x1 = _tc_update(aggp, h, lin2_w[i], lin2_b[i].reshape(1, -1),
                           lin_w[i], lin_b[i].reshape(1, -1), l1n)
    return h


# speedup vs baseline: 1.2843x; 1.2843x over previous
import functools

import jax
import jax.numpy as jnp
import numpy as np
from jax import lax
from jax.experimental import pallas as pl
from jax.experimental.pallas import tpu as pltpu
from jax.experimental.pallas import tpu_sc as plsc

CUTOFF = 10.0
SHIFT = float(np.log(2.0))

NC = 2
NS = 16
NW = NC * NS

K = 80


def _ssp(x):
    return jax.nn.softplus(x) - SHIFT


def _filters_body(ea_ref, el_ref, w1_ref, b1_ref, w2_ref, b2_ref, out_ref):
    ea = ea_ref[...]
    t = jnp.dot(ea, w1_ref[0], preferred_element_type=jnp.float32) + b1_ref[0]
    t = _ssp(t)
    w = jnp.dot(t, w2_ref[0], preferred_element_type=jnp.float32) + b2_ref[0]
    el = el_ref[...]
    c = 0.5 * (jnp.cos(el * (np.pi / CUTOFF)) + 1.0)
    c = c * (el <= CUTOFF).astype(jnp.float32) * (el >= 0.0).astype(jnp.float32)
    out_ref[0] = w * c


def _precompute_filters(edge_attr, edge_length, mlp_w1, mlp_b1, mlp_w2, mlp_b2):
    L, D_EDGE, D = mlp_w1.shape
    E = edge_attr.shape[0]
    BE = 2000
    el2 = edge_length.reshape(E, 1)
    b1 = mlp_b1.reshape(L, 1, D)
    b2 = mlp_b2.reshape(L, 1, D)
    grid = (L, E // BE)
    return pl.pallas_call(
        _filters_body,
        grid=grid,
        in_specs=[
            pl.BlockSpec((BE, D_EDGE), lambda l, e: (e, 0)),
            pl.BlockSpec((BE, 1), lambda l, e: (e, 0)),
            pl.BlockSpec((1, D_EDGE, D), lambda l, e: (l, 0, 0)),
            pl.BlockSpec((1, 1, D), lambda l, e: (l, 0, 0)),
            pl.BlockSpec((1, D, D), lambda l, e: (l, 0, 0)),
            pl.BlockSpec((1, 1, D), lambda l, e: (l, 0, 0)),
        ],
        out_specs=pl.BlockSpec((1, BE, D), lambda l, e: (l, e, 0)),
        out_shape=jax.ShapeDtypeStruct((L, E, D), jnp.float32),
    )(edge_attr, el2, mlp_w1, b1, mlp_w2, b2)


def _matmul_body(x_ref, w_ref, o_ref):
    o_ref[...] = jnp.dot(x_ref[...], w_ref[...], preferred_element_type=jnp.float32)


def _tc_matmul(x, w):
    n, d = x.shape
    BN = 2000
    return pl.pallas_call(
        _matmul_body,
        grid=(n // BN,),
        in_specs=[
            pl.BlockSpec((BN, d), lambda i: (i, 0)),
            pl.BlockSpec((d, d), lambda i: (0, 0)),
        ],
        out_specs=pl.BlockSpec((BN, d), lambda i: (i, 0)),
        out_shape=jax.ShapeDtypeStruct((n, d), jnp.float32),
    )(x, w)


def _update_body(aggp_ref, h_ref, l2w_ref, l2b_ref, lw_ref, lb_ref, l1n_ref,
                 hn_ref, x1_ref):
    agg = aggp_ref[0] + aggp_ref[1]
    t = jnp.dot(agg, l2w_ref[...], preferred_element_type=jnp.float32) + l2b_ref[...]
    t = _ssp(t)
    out = jnp.dot(t, lw_ref[...], preferred_element_type=jnp.float32) + lb_ref[...]
    hn = h_ref[...] + out
    hn_ref[...] = hn
    x1_ref[...] = jnp.dot(hn, l1n_ref[...], preferred_element_type=jnp.float32)


def _tc_update(aggp, h, l2w, l2b, lw, lb, l1n):
    n, d = h.shape
    BN = 2000
    return pl.pallas_call(
        _update_body,
        grid=(n // BN,),
        in_specs=[
            pl.BlockSpec((2, BN, d), lambda i: (0, i, 0)),
            pl.BlockSpec((BN, d), lambda i: (i, 0)),
            pl.BlockSpec((d, d), lambda i: (0, 0)),
            pl.BlockSpec((1, d), lambda i: (0, 0)),
            pl.BlockSpec((d, d), lambda i: (0, 0)),
            pl.BlockSpec((1, d), lambda i: (0, 0)),
            pl.BlockSpec((d, d), lambda i: (0, 0)),
        ],
        out_specs=[
            pl.BlockSpec((BN, d), lambda i: (i, 0)),
            pl.BlockSpec((BN, d), lambda i: (i, 0)),
        ],
        out_shape=[
            jax.ShapeDtypeStruct((n, d), jnp.float32),
            jax.ShapeDtypeStruct((n, d), jnp.float32),
        ],
    )(aggp, h, l2w, l2b, lw, lb, l1n)


def _sc_message(x1, w, src, dst):
    n, d = x1.shape
    e = src.shape[0]
    epw = e // NW
    chunks = epw // K
    nvec = d // 16
    RB = 208
    NCOPY = 3
    rpw = RB * NCOPY
    rem = n - rpw * NS

    mesh = plsc.VectorSubcoreMesh(core_axis_name="c", subcore_axis_name="s")

    @functools.partial(
        pl.kernel,
        out_type=jax.ShapeDtypeStruct((NC, n, d), jnp.float32),
        mesh=mesh,
        scratch_types=[
            pltpu.VMEM((K,), jnp.int32),
            pltpu.VMEM((K,), jnp.int32),
            pltpu.VMEM((K, d), jnp.float32),
            pltpu.VMEM((K, d), jnp.float32),
            pltpu.VMEM((RB, d), jnp.float32),
            pltpu.VMEM_SHARED((n, d), jnp.float32),
            pltpu.SemaphoreType.DMA,
        ],
    )
    def launch(x1_hbm, w_hbm, src_hbm, dst_hbm, out_hbm,
               idxs_v, idxd_v, w_v, x_v, z_v, acc_sh, sem):
        c = lax.axis_index("c")
        s = lax.axis_index("s")
        wid = s * NC + c
        base0 = wid * epw
        row0 = s * rpw

        def _zero(i, _):
            for j in range(nvec):
                z_v[i, pl.ds(j * 16, 16)] = jnp.zeros((16,), jnp.float32)
            return None
        lax.fori_loop(0, RB, _zero, None)

        for k in range(NCOPY):
            pltpu.sync_copy(z_v, acc_sh.at[pl.ds(row0 + k * RB, RB)])

        @pl.when(s == NS - 1)
        def _():
            pltpu.sync_copy(z_v.at[pl.ds(0, rem)],
                            acc_sh.at[pl.ds(rpw * NS, rem)])

        plsc.subcore_barrier()

        def _edge_chunk(g, _):
            base = base0 + g * K
            pltpu.sync_copy(src_hbm.at[pl.ds(base, K)], idxs_v)
            pltpu.sync_copy(dst_hbm.at[pl.ds(base, K)], idxd_v)
            pltpu.async_copy(x1_hbm.at[idxs_v], x_v, sem).wait()
            pltpu.sync_copy(w_hbm.at[pl.ds(base, K)], w_v)

            def _mul(i, _):
                for j in range(nvec):
                    sl = pl.ds(j * 16, 16)
                    x_v[i, sl] = x_v[i, sl] * w_v[i, sl]
                return None
            lax.fori_loop(0, K, _mul, None)

            pltpu.sync_copy(x_v, acc_sh.at[idxd_v], add=True)
            return None
        lax.fori_loop(0, chunks, _edge_chunk, None)

        plsc.subcore_barrier()
        for k in range(NCOPY):
            pltpu.sync_copy(acc_sh.at[pl.ds(row0 + k * RB, RB)],
                            out_hbm.at[c, pl.ds(row0 + k * RB, RB)])

        @pl.when(s == NS - 1)
        def _():
            pltpu.sync_copy(acc_sh.at[pl.ds(rpw * NS, rem)],
                            out_hbm.at[c, pl.ds(rpw * NS, rem)])

    return launch(x1, w, src, dst)


def kernel(z, edge_index, edge_length, edge_attr, mlp_w1, mlp_b1, mlp_w2,
           mlp_b2, lin1_w, lin2_w, lin2_b, lin_w, lin_b):
    L = mlp_w1.shape[0]
    src = edge_index[0]
    dst = edge_index[1]

    w_all = _precompute_filters(edge_attr, edge_length, mlp_w1, mlp_b1,
                                mlp_w2, mlp_b2)

    h = z
    x1 = _tc_matmul(z, lin1_w[0])
    for i in range(L):
        aggp = _sc_message(x1, w_all[i], src, dst)
        l1n = lin1_w[(i + 1) % L]
        h, x1 = _tc_update(aggp, h, lin2_w[i], lin2_b[i].reshape(1, -1),
                           lin_w[i], lin_b[i].reshape(1, -1), l1n)
    return h


# speedup vs baseline: 2.1098x; 1.6428x over previous
import functools

import jax
import jax.numpy as jnp
import numpy as np
from jax import lax
from jax.experimental import pallas as pl
from jax.experimental.pallas import tpu as pltpu
from jax.experimental.pallas import tpu_sc as plsc

CUTOFF = 10.0
SHIFT = float(np.log(2.0))

NC = 2
NS = 16
NW = NC * NS

K = 40


def _ssp(x):
    return jax.nn.softplus(x) - SHIFT


def _filters_body(ea_ref, el_ref, w1_ref, b1_ref, w2_ref, b2_ref, out_ref):
    ea = ea_ref[...]
    t = jnp.dot(ea, w1_ref[0], preferred_element_type=jnp.float32) + b1_ref[0]
    t = _ssp(t)
    w = jnp.dot(t, w2_ref[0], preferred_element_type=jnp.float32) + b2_ref[0]
    el = el_ref[...]
    c = 0.5 * (jnp.cos(el * (np.pi / CUTOFF)) + 1.0)
    c = c * (el <= CUTOFF).astype(jnp.float32) * (el >= 0.0).astype(jnp.float32)
    out_ref[...] = w * c


def _layer_filters(edge_attr, el2, w1, b1, w2, b2):
    D_EDGE, D = w1.shape
    E = edge_attr.shape[0]
    BE = 2000
    return pl.pallas_call(
        _filters_body,
        grid=(E // BE,),
        in_specs=[
            pl.BlockSpec((BE, D_EDGE), lambda e: (e, 0)),
            pl.BlockSpec((BE, 1), lambda e: (e, 0)),
            pl.BlockSpec((1, D_EDGE, D), lambda e: (0, 0, 0)),
            pl.BlockSpec((1, 1, D), lambda e: (0, 0, 0)),
            pl.BlockSpec((1, D, D), lambda e: (0, 0, 0)),
            pl.BlockSpec((1, 1, D), lambda e: (0, 0, 0)),
        ],
        out_specs=pl.BlockSpec((BE, D), lambda e: (e, 0)),
        out_shape=jax.ShapeDtypeStruct((E, D), jnp.float32),
    )(edge_attr, el2, w1.reshape(1, D_EDGE, D), b1.reshape(1, 1, D),
      w2.reshape(1, D, D), b2.reshape(1, 1, D))


def _matmul_body(x_ref, w_ref, o_ref):
    o_ref[...] = jnp.dot(x_ref[...], w_ref[...], preferred_element_type=jnp.float32)


def _tc_matmul(x, w):
    n, d = x.shape
    BN = 2000
    return pl.pallas_call(
        _matmul_body,
        grid=(n // BN,),
        in_specs=[
            pl.BlockSpec((BN, d), lambda i: (i, 0)),
            pl.BlockSpec((d, d), lambda i: (0, 0)),
        ],
        out_specs=pl.BlockSpec((BN, d), lambda i: (i, 0)),
        out_shape=jax.ShapeDtypeStruct((n, d), jnp.float32),
    )(x, w)


def _update_body(aggp_ref, h_ref, l2w_ref, l2b_ref, lw_ref, lb_ref, l1n_ref,
                 hn_ref, x1_ref):
    agg = aggp_ref[0] + aggp_ref[1]
    t = jnp.dot(agg, l2w_ref[...], preferred_element_type=jnp.float32) + l2b_ref[...]
    t = _ssp(t)
    out = jnp.dot(t, lw_ref[...], preferred_element_type=jnp.float32) + lb_ref[...]
    hn = h_ref[...] + out
    hn_ref[...] = hn
    x1_ref[...] = jnp.dot(hn, l1n_ref[...], preferred_element_type=jnp.float32)


def _tc_update(aggp, h, l2w, l2b, lw, lb, l1n):
    n, d = h.shape
    BN = 2000
    return pl.pallas_call(
        _update_body,
        grid=(n // BN,),
        in_specs=[
            pl.BlockSpec((2, BN, d), lambda i: (0, i, 0)),
            pl.BlockSpec((BN, d), lambda i: (i, 0)),
            pl.BlockSpec((d, d), lambda i: (0, 0)),
            pl.BlockSpec((1, d), lambda i: (0, 0)),
            pl.BlockSpec((d, d), lambda i: (0, 0)),
            pl.BlockSpec((1, d), lambda i: (0, 0)),
            pl.BlockSpec((d, d), lambda i: (0, 0)),
        ],
        out_specs=[
            pl.BlockSpec((BN, d), lambda i: (i, 0)),
            pl.BlockSpec((BN, d), lambda i: (i, 0)),
        ],
        out_shape=[
            jax.ShapeDtypeStruct((n, d), jnp.float32),
            jax.ShapeDtypeStruct((n, d), jnp.float32),
        ],
    )(aggp, h, l2w, l2b, lw, lb, l1n)


def _sc_message(x1, w, src, dst):
    n, d = x1.shape
    e = src.shape[0]
    k = K
    epw = e // NW
    chunks = epw // k
    nvec = d // 16
    ZB = 48
    NCOPY = 13
    rpw = ZB * NCOPY
    rem = n - rpw * NS

    mesh = plsc.VectorSubcoreMesh(core_axis_name="c", subcore_axis_name="s")

    @functools.partial(
        pl.kernel,
        out_type=jax.ShapeDtypeStruct((NC, n, d), jnp.float32),
        mesh=mesh,
        scratch_types=[
            pltpu.VMEM((4, k), jnp.int32),
            pltpu.VMEM((4, k), jnp.int32),
            pltpu.VMEM((2, k, d), jnp.float32),
            pltpu.VMEM((3, k, d), jnp.float32),
            pltpu.VMEM((ZB, d), jnp.float32),
            pltpu.VMEM_SHARED((n, d), jnp.float32),
            pltpu.SemaphoreType.DMA,
            pltpu.SemaphoreType.DMA,
            pltpu.SemaphoreType.DMA,
        ],
    )
    def launch(x1_hbm, w_hbm, src_hbm, dst_hbm, out_hbm,
               idxs_v, idxd_v, w_v, x_v, z_v, acc_sh, isem, gsem, wsem):
        c = lax.axis_index("c")
        s = lax.axis_index("s")
        wid = s * NC + c
        base0 = wid * epw
        row0 = s * rpw

        def _zero(i, _):
            for j in range(nvec):
                z_v[i, pl.ds(j * 16, 16)] = jnp.zeros((16,), jnp.float32)
            return None
        lax.fori_loop(0, ZB, _zero, None)

        for kk in range(NCOPY):
            pltpu.sync_copy(z_v, acc_sh.at[pl.ds(row0 + kk * ZB, ZB)])

        @pl.when(s == NS - 1)
        def _():
            pltpu.sync_copy(z_v.at[pl.ds(0, rem)],
                            acc_sh.at[pl.ds(rpw * NS, rem)])

        plsc.subcore_barrier()

        def _start_idx(g):
            b = lax.rem(g, 4)
            pltpu.async_copy(src_hbm.at[pl.ds(base0 + g * k, k)],
                             idxs_v.at[b], isem)
            pltpu.async_copy(dst_hbm.at[pl.ds(base0 + g * k, k)],
                             idxd_v.at[b], isem)

        def _wait_idx():
            pltpu.make_async_copy(src_hbm.at[pl.ds(0, k)], idxs_v.at[0],
                                  isem).wait()
            pltpu.make_async_copy(dst_hbm.at[pl.ds(0, k)], idxd_v.at[0],
                                  isem).wait()

        def _start_fetch(g):
            b = lax.rem(g, 4)
            pltpu.async_copy(x1_hbm.at[idxs_v.at[b]], x_v.at[lax.rem(g, 3)],
                             gsem)
            pltpu.async_copy(w_hbm.at[pl.ds(base0 + g * k, k)],
                             w_v.at[lax.rem(g, 2)], wsem)

        def _wait_fetch():
            pltpu.make_async_copy(x1_hbm.at[idxs_v.at[0]], x_v.at[0],
                                  gsem).wait()
            pltpu.make_async_copy(w_hbm.at[pl.ds(0, k)], w_v.at[0],
                                  wsem).wait()

        _start_idx(0)
        _wait_idx()
        _start_idx(1)
        _start_fetch(0)

        def _edge_chunk(g, _):
            gb = lax.rem(g, 3)
            wb = lax.rem(g, 2)

            _wait_fetch()

            @pl.when(g + 1 < chunks)
            def _():
                _wait_idx()

                @pl.when(g + 2 < chunks)
                def _():
                    _start_idx(g + 2)
                _start_fetch(g + 1)

            def _mul(i, _):
                for j in range(nvec):
                    sl = pl.ds(j * 16, 16)
                    x_v[gb, i, sl] = x_v[gb, i, sl] * w_v[wb, i, sl]
                return None
            lax.fori_loop(0, k, _mul, None)

            pltpu.sync_copy(x_v.at[gb], acc_sh.at[idxd_v.at[lax.rem(g, 4)]],
                            add=True)
            return None
        lax.fori_loop(0, chunks, _edge_chunk, None)

        plsc.subcore_barrier()
        for kk in range(NCOPY):
            pltpu.sync_copy(acc_sh.at[pl.ds(row0 + kk * ZB, ZB)],
                            out_hbm.at[c, pl.ds(row0 + kk * ZB, ZB)])

        @pl.when(s == NS - 1)
        def _():
            pltpu.sync_copy(acc_sh.at[pl.ds(rpw * NS, rem)],
                            out_hbm.at[c, pl.ds(rpw * NS, rem)])

    return launch(x1, w, src, dst)


def kernel(z, edge_index, edge_length, edge_attr, mlp_w1, mlp_b1, mlp_w2,
           mlp_b2, lin1_w, lin2_w, lin2_b, lin_w, lin_b):
    L = mlp_w1.shape[0]
    E = edge_index.shape[1]
    src = edge_index[0]
    dst = edge_index[1]
    el2 = edge_length.reshape(E, 1)

    h = z
    x1 = _tc_matmul(z, lin1_w[0])
    for i in range(L):
        w_i = _layer_filters(edge_attr, el2, mlp_w1[i], mlp_b1[i],
                             mlp_w2[i], mlp_b2[i])
        aggp = _sc_message(x1, w_i, src, dst)
        l1n = lin1_w[(i + 1) % L]
        h, x1 = _tc_update(aggp, h, lin2_w[i], lin2_b[i].reshape(1, -1),
                           lin_w[i], lin_b[i].reshape(1, -1), l1n)
    return h


# speedup vs baseline: 2.2314x; 1.0577x over previous
import functools

import jax
import jax.numpy as jnp
import numpy as np
from jax import lax
from jax.experimental import pallas as pl
from jax.experimental.pallas import tpu as pltpu
from jax.experimental.pallas import tpu_sc as plsc

CUTOFF = 10.0
SHIFT = float(np.log(2.0))

NC = 2
NS = 16
NW = NC * NS

K = 40


def _ssp(x):
    return jax.nn.softplus(x) - SHIFT


def _filters_body(ea_ref, el_ref, w1_ref, b1_ref, w2_ref, b2_ref, out_ref):
    ea = ea_ref[...]
    t = jnp.dot(ea, w1_ref[0], preferred_element_type=jnp.float32) + b1_ref[0]
    t = _ssp(t)
    w = jnp.dot(t, w2_ref[0], preferred_element_type=jnp.float32) + b2_ref[0]
    el = el_ref[...]
    c = 0.5 * (jnp.cos(el * (np.pi / CUTOFF)) + 1.0)
    c = c * (el <= CUTOFF).astype(jnp.float32) * (el >= 0.0).astype(jnp.float32)
    out_ref[...] = w * c


def _layer_filters(edge_attr, el2, w1, b1, w2, b2):
    D_EDGE, D = w1.shape
    E = edge_attr.shape[0]
    BE = 2000
    return pl.pallas_call(
        _filters_body,
        grid=(E // BE,),
        in_specs=[
            pl.BlockSpec((BE, D_EDGE), lambda e: (e, 0)),
            pl.BlockSpec((BE, 1), lambda e: (e, 0)),
            pl.BlockSpec((1, D_EDGE, D), lambda e: (0, 0, 0)),
            pl.BlockSpec((1, 1, D), lambda e: (0, 0, 0)),
            pl.BlockSpec((1, D, D), lambda e: (0, 0, 0)),
            pl.BlockSpec((1, 1, D), lambda e: (0, 0, 0)),
        ],
        out_specs=pl.BlockSpec((BE, D), lambda e: (e, 0)),
        out_shape=jax.ShapeDtypeStruct((E, D), jnp.float32),
    )(edge_attr, el2, w1.reshape(1, D_EDGE, D), b1.reshape(1, 1, D),
      w2.reshape(1, D, D), b2.reshape(1, 1, D))


def _matmul_body(x_ref, w_ref, o_ref):
    o_ref[...] = jnp.dot(x_ref[...], w_ref[...], preferred_element_type=jnp.float32)


def _tc_matmul(x, w):
    n, d = x.shape
    BN = 2000
    return pl.pallas_call(
        _matmul_body,
        grid=(n // BN,),
        in_specs=[
            pl.BlockSpec((BN, d), lambda i: (i, 0)),
            pl.BlockSpec((d, d), lambda i: (0, 0)),
        ],
        out_specs=pl.BlockSpec((BN, d), lambda i: (i, 0)),
        out_shape=jax.ShapeDtypeStruct((n, d), jnp.float32),
    )(x, w)


def _update_body(aggp_ref, h_ref, l2w_ref, l2b_ref, lw_ref, lb_ref, l1n_ref,
                 hn_ref, x1_ref):
    agg = aggp_ref[0] + aggp_ref[1]
    t = jnp.dot(agg, l2w_ref[...], preferred_element_type=jnp.float32) + l2b_ref[...]
    t = _ssp(t)
    out = jnp.dot(t, lw_ref[...], preferred_element_type=jnp.float32) + lb_ref[...]
    hn = h_ref[...] + out
    hn_ref[...] = hn
    x1_ref[...] = jnp.dot(hn, l1n_ref[...], preferred_element_type=jnp.float32)


def _tc_update(aggp, h, l2w, l2b, lw, lb, l1n):
    n, d = h.shape
    BN = 2000
    return pl.pallas_call(
        _update_body,
        grid=(n // BN,),
        in_specs=[
            pl.BlockSpec((2, BN, d), lambda i: (0, i, 0)),
            pl.BlockSpec((BN, d), lambda i: (i, 0)),
            pl.BlockSpec((d, d), lambda i: (0, 0)),
            pl.BlockSpec((1, d), lambda i: (0, 0)),
            pl.BlockSpec((d, d), lambda i: (0, 0)),
            pl.BlockSpec((1, d), lambda i: (0, 0)),
            pl.BlockSpec((d, d), lambda i: (0, 0)),
        ],
        out_specs=[
            pl.BlockSpec((BN, d), lambda i: (i, 0)),
            pl.BlockSpec((BN, d), lambda i: (i, 0)),
        ],
        out_shape=[
            jax.ShapeDtypeStruct((n, d), jnp.float32),
            jax.ShapeDtypeStruct((n, d), jnp.float32),
        ],
    )(aggp, h, l2w, l2b, lw, lb, l1n)


def _sc_message(x1, w, src, dst):
    n, d = x1.shape
    e = src.shape[0]
    k = K
    epw = e // NW
    chunks = epw // k
    nvec = d // 16
    ZB = 48
    NCOPY = 13
    rpw = ZB * NCOPY
    rem = n - rpw * NS

    mesh = plsc.VectorSubcoreMesh(core_axis_name="c", subcore_axis_name="s")

    @functools.partial(
        pl.kernel,
        out_type=jax.ShapeDtypeStruct((NC, n, d), jnp.float32),
        mesh=mesh,
        scratch_types=[
            pltpu.VMEM((4, k), jnp.int32),
            pltpu.VMEM((4, k), jnp.int32),
            pltpu.VMEM((2, k, d), jnp.float32),
            pltpu.VMEM((3, k, d), jnp.float32),
            pltpu.VMEM((ZB, d), jnp.float32),
            pltpu.VMEM_SHARED((n, d), jnp.float32),
            pltpu.SemaphoreType.DMA,
            pltpu.SemaphoreType.DMA,
            pltpu.SemaphoreType.DMA,
        ],
    )
    def launch(x1_hbm, w_hbm, src_hbm, dst_hbm, out_hbm,
               idxs_v, idxd_v, w_v, x_v, z_v, acc_sh, isem, gsem, wsem):
        c = lax.axis_index("c")
        s = lax.axis_index("s")
        wid = s * NC + c
        base0 = wid * epw
        row0 = s * rpw

        def _zero(i, _):
            for j in range(nvec):
                z_v[i, pl.ds(j * 16, 16)] = jnp.zeros((16,), jnp.float32)
            return None
        lax.fori_loop(0, ZB, _zero, None)

        for kk in range(NCOPY):
            pltpu.sync_copy(z_v, acc_sh.at[pl.ds(row0 + kk * ZB, ZB)])

        @pl.when(s == NS - 1)
        def _():
            pltpu.sync_copy(z_v.at[pl.ds(0, rem)],
                            acc_sh.at[pl.ds(rpw * NS, rem)])

        plsc.subcore_barrier()

        def _start_idx(g):
            b = lax.rem(g, 4)
            pltpu.async_copy(src_hbm.at[pl.ds(base0 + g * k, k)],
                             idxs_v.at[b], isem)
            pltpu.async_copy(dst_hbm.at[pl.ds(base0 + g * k, k)],
                             idxd_v.at[b], isem)

        def _wait_idx():
            pltpu.make_async_copy(src_hbm.at[pl.ds(0, k)], idxs_v.at[0],
                                  isem).wait()
            pltpu.make_async_copy(dst_hbm.at[pl.ds(0, k)], idxd_v.at[0],
                                  isem).wait()

        def _start_fetch(g):
            b = lax.rem(g, 4)
            pltpu.async_copy(x1_hbm.at[idxs_v.at[b]], x_v.at[lax.rem(g, 3)],
                             gsem)
            pltpu.async_copy(w_hbm.at[pl.ds(base0 + g * k, k)],
                             w_v.at[lax.rem(g, 2)], wsem)

        def _wait_fetch():
            pltpu.make_async_copy(x1_hbm.at[idxs_v.at[0]], x_v.at[0],
                                  gsem).wait()
            pltpu.make_async_copy(w_hbm.at[pl.ds(0, k)], w_v.at[0],
                                  wsem).wait()

        _start_idx(0)
        _wait_idx()
        _start_idx(1)
        _start_fetch(0)

        def _edge_chunk(g, _):
            gb = lax.rem(g, 3)
            wb = lax.rem(g, 2)

            _wait_fetch()

            @pl.when(g + 1 < chunks)
            def _():
                _wait_idx()

                @pl.when(g + 2 < chunks)
                def _():
                    _start_idx(g + 2)
                _start_fetch(g + 1)

            @plsc.parallel_loop(0, k, unroll=4)
            def _(i):
                for j in range(nvec):
                    sl = pl.ds(j * 16, 16)
                    x_v[gb, i, sl] = x_v[gb, i, sl] * w_v[wb, i, sl]

            pltpu.sync_copy(x_v.at[gb], acc_sh.at[idxd_v.at[lax.rem(g, 4)]],
                            add=True)
            return None
        lax.fori_loop(0, chunks, _edge_chunk, None)

        plsc.subcore_barrier()
        for kk in range(NCOPY):
            pltpu.sync_copy(acc_sh.at[pl.ds(row0 + kk * ZB, ZB)],
                            out_hbm.at[c, pl.ds(row0 + kk * ZB, ZB)])

        @pl.when(s == NS - 1)
        def _():
            pltpu.sync_copy(acc_sh.at[pl.ds(rpw * NS, rem)],
                            out_hbm.at[c, pl.ds(rpw * NS, rem)])

    return launch(x1, w, src, dst)


def kernel(z, edge_index, edge_length, edge_attr, mlp_w1, mlp_b1, mlp_w2,
           mlp_b2, lin1_w, lin2_w, lin2_b, lin_w, lin_b):
    L = mlp_w1.shape[0]
    E = edge_index.shape[1]
    src = edge_index[0]
    dst = edge_index[1]
    el2 = edge_length.reshape(E, 1)

    h = z
    x1 = _tc_matmul(z, lin1_w[0])
    for i in range(L):
        w_i = _layer_filters(edge_attr, el2, mlp_w1[i], mlp_b1[i],
                             mlp_w2[i], mlp_b2[i])
        aggp = _sc_message(x1, w_i, src, dst)
        l1n = lin1_w[(i + 1) % L]
        h, x1 = _tc_update(aggp, h, lin2_w[i], lin2_b[i].reshape(1, -1),
                           lin_w[i], lin_b[i].reshape(1, -1), l1n)
    return h
